# baseline (speedup 1.0000x reference)
import functools

import jax
import jax.numpy as jnp
from jax import lax
from jax.experimental import pallas as pl
from jax.experimental.pallas import tpu as pltpu
from jax.experimental.pallas import tpu_sc as plsc

B, N, C, M = 8, 50000, 128, 25000
NC, NS = 2, 16
W = NC * NS
WPB = W // B
K = 192
STRIDE = 6272
NFULL = 32
TAIL_A = STRIDE - NFULL * K
TAIL_Q3 = M - (WPB - 1) * STRIDE - NFULL * K
NBUF = 4
WIN = STRIDE


def _gather_body(x_hbm, idx_hbm, out_hbm, idx_v, bufs, gsems, wsems):
    wid = lax.axis_index("s") * NC + lax.axis_index("c")
    b = wid // WPB
    q = wid % WPB
    mbase = q * STRIDE

    start = b * M + q * STRIDE
    off = jnp.where(wid == W - 1, start - (B * M - WIN), 0)
    pltpu.sync_copy(idx_hbm.at[pl.ds(start - off, WIN)], idx_v)

    def start_gather(ci, j):
        pltpu.async_copy(
            x_hbm.at[b].at[idx_v.at[pl.ds(off + ci * K, K)]],
            bufs.at[j],
            gsems.at[j],
        )

    def start_write(ci, j):
        pltpu.async_copy(
            bufs.at[j], out_hbm.at[b].at[pl.ds(mbase + ci * K, K)], wsems.at[j]
        )

    def drain(sems, j, rows=K):
        pltpu.make_async_copy(
            x_hbm.at[b].at[pl.ds(0, rows)], bufs.at[j, pl.ds(0, rows)], sems.at[j]
        ).wait()

    for j in range(NBUF):
        start_gather(j, j)

    def body(i, carry):
        for j in range(NBUF):
            drain(gsems, j)
            start_write(NBUF * i + j, j)
        for j in range(NBUF):
            drain(wsems, j)
            start_gather(NBUF * i + j + NBUF, j)
        return carry

    lax.fori_loop(0, NFULL // NBUF - 1, body, 0)

    last = NFULL - NBUF
    for j in range(NBUF):
        drain(gsems, j)
        start_write(last + j, j)
    drain(wsems, 0)
    pltpu.async_copy(
        x_hbm.at[b].at[idx_v.at[pl.ds(WIN - K, K)]], bufs.at[0], gsems.at[0]
    )

    drain(gsems, 0)

    @pl.when(q < WPB - 1)
    def _():
        pltpu.sync_copy(
            bufs.at[0, pl.ds(K - TAIL_A, TAIL_A)],
            out_hbm.at[b].at[pl.ds(mbase + NFULL * K, TAIL_A)],
        )

    @pl.when(q == WPB - 1)
    def _():
        pltpu.sync_copy(
            bufs.at[0, pl.ds(K - TAIL_A + off, TAIL_Q3)],
            out_hbm.at[b].at[pl.ds(mbase + NFULL * K, TAIL_Q3)],
        )

    for j in range(1, NBUF):
        drain(wsems, j)


@functools.partial(jax.jit, static_argnames=("interpret",))
def kernel(x, pool_idx, interpret=False):
    mesh = plsc.VectorSubcoreMesh(
        core_axis_name="c", subcore_axis_name="s", num_cores=NC, num_subcores=NS
    )
    run = pl.kernel(
        _gather_body,
        out_type=jax.ShapeDtypeStruct((B, M, C), jnp.float32),
        mesh=mesh,
        scratch_types=[
            pltpu.VMEM((WIN,), jnp.int32),
            pltpu.VMEM((NBUF, K, C), jnp.float32),
            pltpu.SemaphoreType.DMA((NBUF,)),
            pltpu.SemaphoreType.DMA((NBUF,)),
        ],
        interpret=interpret,
    )
    return run(x, pool_idx.reshape(B * M))


# speedup vs baseline: 1.0557x; 1.0557x over previous
import functools

import jax
import jax.numpy as jnp
from jax import lax
from jax.experimental import pallas as pl
from jax.experimental.pallas import tpu as pltpu
from jax.experimental.pallas import tpu_sc as plsc

B, N, C, M = 8, 50000, 128, 25000
NC, NS = 2, 16
W = NC * NS
WPB = W // B
K = 128
STRIDE = 6272
NCHUNK = 49
NFULL = NCHUNK - 1
TAIL_Q3 = M - (WPB - 1) * STRIDE - NFULL * K
NBUF = 6
WIN = NCHUNK * K


def _gather_body(x_hbm, idx_hbm, out_hbm, idx_v, bufs, gsems, wsems):
    wid = lax.axis_index("s") * NC + lax.axis_index("c")
    b = wid // WPB
    q = wid % WPB
    mbase = q * STRIDE

    start = b * M + q * STRIDE
    off = jnp.where(wid == W - 1, start - (B * M - WIN), 0)
    pltpu.sync_copy(idx_hbm.at[pl.ds(start - off, WIN)], idx_v)

    def start_gather(ci, j):
        pltpu.async_copy(
            x_hbm.at[b].at[idx_v.at[pl.ds(off + ci * K, K)]],
            bufs.at[j],
            gsems.at[j],
        )

    def start_write(ci, j):
        pltpu.async_copy(
            bufs.at[j], out_hbm.at[b].at[pl.ds(mbase + ci * K, K)], wsems.at[j]
        )

    def drain(sems, j, rows=K):
        pltpu.make_async_copy(
            x_hbm.at[b].at[pl.ds(0, rows)], bufs.at[j, pl.ds(0, rows)], sems.at[j]
        ).wait()

    for j in range(NBUF):
        start_gather(j, j)

    def body(i, carry):
        for j in range(NBUF):
            drain(gsems, j)
            start_write(NBUF * i + j, j)
        for j in range(NBUF):
            drain(wsems, j)
            start_gather(NBUF * i + j + NBUF, j)
        return carry

    lax.fori_loop(0, NFULL // NBUF - 1, body, 0)

    last = NFULL - NBUF
    for j in range(NBUF):
        drain(gsems, j)
        start_write(last + j, j)
    drain(wsems, 0)
    pltpu.async_copy(
        x_hbm.at[b].at[idx_v.at[pl.ds(WIN - K, K)]], bufs.at[0], gsems.at[0]
    )

    drain(gsems, 0)

    @pl.when(q < WPB - 1)
    def _():
        pltpu.sync_copy(bufs.at[0], out_hbm.at[b].at[pl.ds(mbase + NFULL * K, K)])

    @pl.when(q == WPB - 1)
    def _():
        pltpu.sync_copy(
            bufs.at[0, pl.ds(off, TAIL_Q3)],
            out_hbm.at[b].at[pl.ds(mbase + NFULL * K, TAIL_Q3)],
        )

    for j in range(1, NBUF):
        drain(wsems, j)


@functools.partial(jax.jit, static_argnames=("interpret",))
def kernel(x, pool_idx, interpret=False):
    mesh = plsc.VectorSubcoreMesh(
        core_axis_name="c", subcore_axis_name="s", num_cores=NC, num_subcores=NS
    )
    run = pl.kernel(
        _gather_body,
        out_type=jax.ShapeDtypeStruct((B, M, C), jnp.float32),
        mesh=mesh,
        scratch_types=[
            pltpu.VMEM((WIN,), jnp.int32),
            pltpu.VMEM((NBUF, K, C), jnp.float32),
            pltpu.SemaphoreType.DMA((NBUF,)),
            pltpu.SemaphoreType.DMA((NBUF,)),
        ],
        interpret=interpret,
    )
    return run(x, pool_idx.reshape(B * M))
